# Initial kernel scaffold; baseline (speedup 1.0000x reference)
import jax
import jax.numpy as jnp
from jax.experimental import pallas as pl


def kernel(coordinates, features, W_l1, b_l1, W_l2, b_l2, W_q, W_k, W_v, W_m1, b_m1, W_m2, b_m2, W_p1, b_p1, W_p2, b_p2):
    raise NotImplementedError("write your pallas kernel here")


# speedup vs baseline: 15.8729x; 15.8729x over previous
import functools
import math

import jax
import jax.numpy as jnp
from jax import lax
from jax.experimental import pallas as pl
from jax.experimental.pallas import tpu as pltpu
from jax.experimental.pallas import tpu_sc as plsc

B, N, D, K = 4, 2048, 128, 16
BLK = 256
NBLK = N // BLK
CP = 16
R = B * N * K
NW = 32
RPW = R // NW
CHUNK = 128
NCHUNK = RPW // CHUNK


def _proj_knn_body(coordsT_ref, coords_ref, feats_ref,
                   wl1_ref, bl1_ref, wq_ref, wk_ref, wv_ref, wp1_ref,
                   q_ref, xk_ref, xv_ref, u_ref, idx_ref):
    b = pl.program_id(0)

    feats = feats_ref[0]
    x = jnp.dot(feats, wl1_ref[...], preferred_element_type=jnp.float32)
    x = x + bl1_ref[...]
    q_ref[0] = jnp.dot(x, wq_ref[...], preferred_element_type=jnp.float32)
    xk_ref[0] = jnp.dot(x, wk_ref[...], preferred_element_type=jnp.float32)
    xv_ref[0] = jnp.dot(x, wv_ref[...], preferred_element_type=jnp.float32)

    coords = coords_ref[0]
    c16 = jnp.concatenate(
        [coords, jnp.zeros((BLK, CP - 3), jnp.float32)], axis=1)
    u_ref[0] = jnp.dot(c16, wp1_ref[...], preferred_element_type=jnp.float32)

    dx = coords[:, 0:1] - coordsT_ref[0, 0:1, :]
    dy = coords[:, 1:2] - coordsT_ref[0, 1:2, :]
    dz = coords[:, 2:3] - coordsT_ref[0, 2:3, :]
    d = (dx * dx + dy * dy) + dz * dz

    iota = lax.broadcasted_iota(jnp.int32, (BLK, N), 1)
    inf = jnp.float32(jnp.inf)
    cols = []
    for _ in range(K):
        m = jnp.min(d, axis=1, keepdims=True)
        cand = jnp.where(d == m, iota, jnp.int32(N))
        j = jnp.min(cand, axis=1, keepdims=True)
        cols.append(j)
        d = jnp.where(iota == j, inf, d)
    idx_ref[0] = jnp.concatenate(cols, axis=1) + b * N


def _proj_knn(coordinates, features, wl1t, bl1, wqt, wkt, wvt, wp1t):
    grid = (B, NBLK)
    coordsT = jnp.transpose(coordinates, (0, 2, 1))
    return pl.pallas_call(
        _proj_knn_body,
        grid=grid,
        in_specs=[
            pl.BlockSpec((1, 3, N), lambda b, nb: (b, 0, 0)),
            pl.BlockSpec((1, BLK, 3), lambda b, nb: (b, nb, 0)),
            pl.BlockSpec((1, BLK, D), lambda b, nb: (b, nb, 0)),
            pl.BlockSpec((D, D), lambda b, nb: (0, 0)),
            pl.BlockSpec((1, D), lambda b, nb: (0, 0)),
            pl.BlockSpec((D, D), lambda b, nb: (0, 0)),
            pl.BlockSpec((D, D), lambda b, nb: (0, 0)),
            pl.BlockSpec((D, D), lambda b, nb: (0, 0)),
            pl.BlockSpec((CP, D), lambda b, nb: (0, 0)),
        ],
        out_specs=[
            pl.BlockSpec((1, BLK, D), lambda b, nb: (b, nb, 0)),
            pl.BlockSpec((1, BLK, D), lambda b, nb: (b, nb, 0)),
            pl.BlockSpec((1, BLK, D), lambda b, nb: (b, nb, 0)),
            pl.BlockSpec((1, BLK, D), lambda b, nb: (b, nb, 0)),
            pl.BlockSpec((1, BLK, K), lambda b, nb: (b, nb, 0)),
        ],
        out_shape=[
            jax.ShapeDtypeStruct((B, N, D), jnp.float32),
            jax.ShapeDtypeStruct((B, N, D), jnp.float32),
            jax.ShapeDtypeStruct((B, N, D), jnp.float32),
            jax.ShapeDtypeStruct((B, N, D), jnp.float32),
            jax.ShapeDtypeStruct((B, N, K), jnp.int32),
        ],
    )(coordsT, coordinates, features, wl1t, bl1, wqt, wkt, wvt, wp1t)


def _sc_gather(idx_flat, xk_flat, xv_flat, u_flat):
    mesh = plsc.VectorSubcoreMesh(core_axis_name="c", subcore_axis_name="s")

    @functools.partial(
        pl.kernel,
        mesh=mesh,
        out_type=[
            jax.ShapeDtypeStruct((R, D), jnp.float32),
            jax.ShapeDtypeStruct((R, D), jnp.float32),
            jax.ShapeDtypeStruct((R, D), jnp.float32),
        ],
        scratch_types=[
            pltpu.VMEM((CHUNK,), jnp.int32),
            pltpu.VMEM((CHUNK, D), jnp.float32),
            pltpu.VMEM((CHUNK, D), jnp.float32),
            pltpu.VMEM((CHUNK, D), jnp.float32),
            pltpu.SemaphoreType.DMA,
            pltpu.SemaphoreType.DMA,
            pltpu.SemaphoreType.DMA,
        ],
    )
    def gather_kernel(idx_hbm, xk_hbm, xv_hbm, u_hbm,
                      kg_hbm, vg_hbm, ug_hbm,
                      idx_v, kbuf, vbuf, ubuf, sk, sv, su):
        wid = lax.axis_index("s") * 2 + lax.axis_index("c")

        def body(i, carry):
            base = wid * RPW + i * CHUNK
            pltpu.sync_copy(idx_hbm.at[pl.ds(base, CHUNK)], idx_v)
            ck = pltpu.async_copy(xk_hbm.at[idx_v], kbuf, sk)
            cv = pltpu.async_copy(xv_hbm.at[idx_v], vbuf, sv)
            cu = pltpu.async_copy(u_hbm.at[idx_v], ubuf, su)
            ck.wait()
            cv.wait()
            cu.wait()
            pltpu.sync_copy(kbuf, kg_hbm.at[pl.ds(base, CHUNK)])
            pltpu.sync_copy(vbuf, vg_hbm.at[pl.ds(base, CHUNK)])
            pltpu.sync_copy(ubuf, ug_hbm.at[pl.ds(base, CHUNK)])
            return carry

        lax.fori_loop(0, NCHUNK, body, 0)

    return gather_kernel(idx_flat, xk_flat, xv_flat, u_flat)


def _attn_body(q_ref, feats_ref, ui_ref, kg_ref, vg_ref, ug_ref,
               bp1_ref, wp2_ref, bp2_ref,
               wm1_ref, bm1_ref, wm2_ref, bm2_ref,
               wl2_ref, bl2_ref,
               attn_ref, out_ref):
    q = q_ref[0]
    kg = kg_ref[0]
    vg = vg_ref[0]
    ug = ug_ref[0]
    ui = ui_ref[0]

    p13 = jnp.maximum(
        (ui + bp1_ref[...])[:, None, :] - ug.reshape(BLK, K, D), 0.0)
    p1 = p13.reshape(BLK * K, D)
    pe = jnp.dot(p1, wp2_ref[...], preferred_element_type=jnp.float32)
    pe = pe + bp2_ref[...]
    pe3 = pe.reshape(BLK, K, D)

    t3 = q[:, None, :] - kg.reshape(BLK, K, D) + pe3
    t = t3.reshape(BLK * K, D)
    a1 = jnp.dot(t, wm1_ref[...], preferred_element_type=jnp.float32)
    a1 = jnp.maximum(a1 + bm1_ref[...], 0.0)
    lg = jnp.dot(a1, wm2_ref[...], preferred_element_type=jnp.float32)
    lg = (lg + bm2_ref[...]) / math.sqrt(D)

    s3 = lg.reshape(BLK, K, D)
    mx = jnp.max(s3, axis=1, keepdims=True)
    e3 = jnp.exp(s3 - mx)
    ssum = jnp.sum(e3, axis=1, keepdims=True)
    attn3 = e3 / ssum
    attn_ref[0] = attn3.reshape(BLK * K, D)

    w3 = attn3 * (vg.reshape(BLK, K, D) + pe3)
    o = jnp.sum(w3, axis=1)
    o = jnp.dot(o, wl2_ref[...], preferred_element_type=jnp.float32)
    out_ref[0] = o + bl2_ref[...] + feats_ref[0]


def _attn(q, features, u, kg, vg, ug,
          bp1, wp2t, bp2, wm1t, bm1, wm2t, bm2, wl2t, bl2):
    G = B * NBLK
    qv = q.reshape(G, BLK, D)
    fv = features.reshape(G, BLK, D)
    uiv = u.reshape(G, BLK, D)
    kgv = kg.reshape(G, BLK * K, D)
    vgv = vg.reshape(G, BLK * K, D)
    ugv = ug.reshape(G, BLK * K, D)
    full = lambda shape: pl.BlockSpec((1,) + shape, lambda g: (g, 0, 0))
    wspec = lambda shape: pl.BlockSpec(shape, lambda g: (0,) * len(shape))
    attn_f, out = pl.pallas_call(
        _attn_body,
        grid=(G,),
        in_specs=[
            full((BLK, D)), full((BLK, D)), full((BLK, D)),
            full((BLK * K, D)), full((BLK * K, D)), full((BLK * K, D)),
            wspec((1, D)), wspec((D, D)), wspec((1, D)),
            wspec((D, D)), wspec((1, D)), wspec((D, D)), wspec((1, D)),
            wspec((D, D)), wspec((1, D)),
        ],
        out_specs=[
            full((BLK * K, D)),
            full((BLK, D)),
        ],
        out_shape=[
            jax.ShapeDtypeStruct((G, BLK * K, D), jnp.float32),
            jax.ShapeDtypeStruct((G, BLK, D), jnp.float32),
        ],
    )(qv, fv, uiv, kgv, vgv, ugv,
      bp1, wp2t, bp2, wm1t, bm1, wm2t, bm2, wl2t, bl2)
    return attn_f.reshape(B, N, K, D), out.reshape(B, N, D)


def kernel(coordinates, features, W_l1, b_l1, W_l2, b_l2, W_q, W_k, W_v,
           W_m1, b_m1, W_m2, b_m2, W_p1, b_p1, W_p2, b_p2):
    wl1t = W_l1.T
    wqt = W_q.T
    wkt = W_k.T
    wvt = W_v.T
    wm1t = W_m1.T
    wm2t = W_m2.T
    wl2t = W_l2.T
    wp2t = W_p2.T
    wp1t = jnp.zeros((CP, D), jnp.float32).at[0:3].set(W_p1.T)
    bl1 = b_l1.reshape(1, D)
    bl2 = b_l2.reshape(1, D)
    bm1 = b_m1.reshape(1, D)
    bm2 = b_m2.reshape(1, D)
    bp1 = b_p1.reshape(1, D)
    bp2 = b_p2.reshape(1, D)

    q, xk, xv, u, idx = _proj_knn(coordinates, features, wl1t, bl1,
                                  wqt, wkt, wvt, wp1t)

    kg, vg, ug = _sc_gather(idx.reshape(R),
                            xk.reshape(B * N, D),
                            xv.reshape(B * N, D),
                            u.reshape(B * N, D))

    attn, out = _attn(q, features, u, kg, vg, ug,
                      bp1, wp2t, bp2, wm1t, bm1, wm2t, bm2,
                      wl2t, bl2)
    return (out, attn)
